# baseline (speedup 1.0000x reference)
import functools

import jax
import jax.numpy as jnp
from jax import lax
from jax.experimental import pallas as pl
from jax.experimental.pallas import tpu as pltpu
from jax.experimental.pallas import tpu_sc as plsc

B = 4096
F = 26
V = 100000
E = 16
L1 = 256
L2 = 64

NW = 32
N = B * F
N_PER_W = N // NW
CHUNK = 128
N_CHUNKS = N_PER_W // CHUNK

@functools.cache
def _make_sc_gather():
    mesh = plsc.VectorSubcoreMesh(core_axis_name="c", subcore_axis_name="s")
    return functools.partial(
        pl.kernel,
        out_type=jax.ShapeDtypeStruct((N, E), jnp.float32),
        mesh=mesh,
        scratch_types=[
            pltpu.VMEM((N_PER_W,), jnp.int32),
            pltpu.VMEM((N_PER_W, E), jnp.float32),
            pltpu.SemaphoreType.DMA,
        ],
        compiler_params=pltpu.CompilerParams(use_tc_tiling_on_sc=False),
    )(_sc_gather_body)


def _sc_gather_body(tab_hbm, idx_hbm, out_hbm, idx_v, rows_v, sem):
    wid = lax.axis_index("c") * 16 + lax.axis_index("s")
    base = wid * N_PER_W
    pltpu.sync_copy(idx_hbm.at[pl.ds(base, N_PER_W)], idx_v)

    lane = lax.iota(jnp.int32, 16)

    def _add_offsets(t, carry):
        n = (base + t * 16) + lane
        f = n % F
        idx_v[pl.ds(t * 16, 16)] = idx_v[pl.ds(t * 16, 16)] + f * V
        return carry

    lax.fori_loop(0, N_PER_W // 16, _add_offsets, 0)

    copies = []
    for j in range(N_CHUNKS):
        copies.append(
            pltpu.async_copy(
                tab_hbm.at[idx_v.at[pl.ds(j * CHUNK, CHUNK)]],
                rows_v.at[pl.ds(j * CHUNK, CHUNK)],
                sem,
            )
        )
    for c in copies:
        c.wait()

    pltpu.sync_copy(rows_v, out_hbm.at[pl.ds(base, N_PER_W)])


def _dense_body(x_ref, w1_ref, b1_ref, w2_ref, b2_ref, p_ref, out_ref):
    x = x_ref[...]
    h1 = jnp.dot(x, w1_ref[...], preferred_element_type=jnp.float32)
    h1 = jnp.maximum(h1 + b1_ref[...], 0.0)
    ux = jnp.dot(h1, w2_ref[...], preferred_element_type=jnp.float32)
    ux = jnp.maximum(ux + b2_ref[...], 0.0)
    mx_ = jnp.dot(ux, p_ref[...], preferred_element_type=jnp.float32)
    m = jnp.max(mx_, axis=-1, keepdims=True)
    ex = jnp.exp(mx_ - m)
    mx = (jnp.float32(F) * ex) / jnp.sum(ex, axis=-1, keepdims=True)

    j_ids = lax.broadcasted_iota(jnp.int32, (F, F * E), 1)
    f_ids = lax.broadcasted_iota(jnp.int32, (F, F * E), 0)
    st = jnp.where(j_ids // E == f_ids, 1.0, 0.0).astype(jnp.float32)
    mxr = jnp.dot(mx, st, preferred_element_type=jnp.float32)

    a = mxr * x
    jj = lax.broadcasted_iota(jnp.int32, (F * E, E), 0)
    ee = lax.broadcasted_iota(jnp.int32, (F * E, E), 1)
    s = jnp.where(jj % E == ee, 1.0, 0.0).astype(jnp.float32)
    sum_v = jnp.dot(a, s, preferred_element_type=jnp.float32)
    sum_sq = jnp.dot(a * a, s, preferred_element_type=jnp.float32)
    out_ref[...] = 0.5 * jnp.sum(sum_v * sum_v - sum_sq, axis=-1, keepdims=True)


_BB = 1024


def _dense(x, W1, b1, W2, b2, P):
    grid = (B // _BB,)
    return pl.pallas_call(
        _dense_body,
        grid=grid,
        in_specs=[
            pl.BlockSpec((_BB, F * E), lambda i: (i, 0)),
            pl.BlockSpec((F * E, L1), lambda i: (0, 0)),
            pl.BlockSpec((1, L1), lambda i: (0, 0)),
            pl.BlockSpec((L1, L2), lambda i: (0, 0)),
            pl.BlockSpec((1, L2), lambda i: (0, 0)),
            pl.BlockSpec((L2, F), lambda i: (0, 0)),
        ],
        out_specs=pl.BlockSpec((_BB, 1), lambda i: (i, 0)),
        out_shape=jax.ShapeDtypeStruct((B, 1), jnp.float32),
    )(x, W1, b1, W2, b2, P)


def kernel(sparse, dense, tables, W1, b1, W2, b2, P):
    tab = tables.reshape(F * V, E)
    idx = sparse.reshape(N)
    emb_flat = _make_sc_gather()(tab, idx)
    x = emb_flat.reshape(B, F * E)
    return _dense(x, W1, b1.reshape(1, L1), W2, b2.reshape(1, L2), P)
